# Initial kernel scaffold; baseline (speedup 1.0000x reference)
import jax
import jax.numpy as jnp
from jax.experimental import pallas as pl


def kernel(inputs, table, W, b):
    raise NotImplementedError("write your pallas kernel here")


# speedup vs baseline: 4.8159x; 4.8159x over previous
import functools

import jax
import jax.numpy as jnp
from jax import lax
from jax.experimental import pallas as pl
from jax.experimental.pallas import tpu as pltpu
from jax.experimental.pallas import tpu_sc as plsc

B = 16384
S = 200
V = 10
D = 6
M = B * S
L = 16

NC = 2
NS = 16
NW = NC * NS
PW = M // NW
CHUNK = 4096
NCHUNK = PW // CHUNK


def _sc_seq_body(idx_hbm, tab_hbm, out_hbm, idx_v, out_v, tab_v):
    wid = lax.axis_index("s") * NC + lax.axis_index("c")
    base = wid * PW
    pltpu.sync_copy(tab_hbm, tab_v)
    iota6 = lax.iota(jnp.int32, L) * D

    def chunk_body(k, carry):
        tok0 = base + k * CHUNK
        pltpu.sync_copy(idx_hbm.at[pl.ds(tok0, CHUNK)], idx_v)

        def inner(g, c):
            e = idx_v[pl.ds(g * L, L)]
            pos = iota6 + g * (L * D)
            for d in range(D):
                vals = plsc.load_gather(tab_v, [e + d * L])
                plsc.store_scatter(out_v, [pos + d], vals)
            return c

        lax.fori_loop(0, CHUNK // L, inner, 0, unroll=2)
        pltpu.sync_copy(out_v, out_hbm.at[pl.ds(tok0 * D, CHUNK * D)])
        return carry

    lax.fori_loop(0, NCHUNK, chunk_body, 0)


_sc_seq = functools.partial(
    pl.kernel,
    mesh=plsc.VectorSubcoreMesh(core_axis_name="c", subcore_axis_name="s"),
    out_type=jax.ShapeDtypeStruct((M * D,), jnp.float32),
    scratch_types=[
        pltpu.VMEM((CHUNK,), jnp.int32),
        pltpu.VMEM((CHUNK * D,), jnp.float32),
        pltpu.VMEM((128,), jnp.float32),
    ],
    compiler_params=pltpu.CompilerParams(needs_layout_passes=False),
)(_sc_seq_body)


_PB = 1024


def _pool_body(idx_ref, tp_ref, wp_ref, bp_ref, out_ref):
    n = jnp.dot(tp_ref[...], wp_ref[...], preferred_element_type=jnp.float32)
    n = n + bp_ref[...]
    idx = idx_ref[...]
    acc = jnp.zeros((_PB, D), jnp.float32)
    for v in range(V):
        acc = jnp.where(idx == v, n[v, :D], acc)
    out_ref[...] = jnp.tanh(acc)


def _pooled(idx0, tp, wp, bp):
    return pl.pallas_call(
        _pool_body,
        grid=(B // _PB,),
        in_specs=[
            pl.BlockSpec((_PB, 1), lambda i: (i, 0)),
            pl.BlockSpec((16, 128), lambda i: (0, 0)),
            pl.BlockSpec((128, 128), lambda i: (0, 0)),
            pl.BlockSpec((16, 128), lambda i: (0, 0)),
        ],
        out_specs=pl.BlockSpec((_PB, D), lambda i: (i, 0)),
        out_shape=jax.ShapeDtypeStruct((B, D), jnp.float32),
    )(idx0, tp, wp, bp)


def kernel(inputs, table, W, b):
    idx_flat = inputs.reshape(M)
    tab1d = jnp.zeros((8, L), jnp.float32).at[:D, :V].set(table.T).reshape(128)
    seq_flat = _sc_seq(idx_flat, tab1d)
    seq = seq_flat.reshape(B, S, D)

    idx0 = inputs[:, :1]
    tp = jnp.zeros((16, 128), jnp.float32).at[:V, :D].set(table)
    wp = jnp.zeros((128, 128), jnp.float32).at[:D, :D].set(W)
    bp = jnp.zeros((16, 128), jnp.float32).at[:, :D].set(b)
    pooled = _pooled(idx0, tp, wp, bp)
    return (seq, pooled)


# speedup vs baseline: 25.2342x; 5.2398x over previous
import functools

import jax
import jax.numpy as jnp
from jax import lax
from jax.experimental import pallas as pl
from jax.experimental.pallas import tpu as pltpu
from jax.experimental.pallas import tpu_sc as plsc

B = 16384
S = 200
V = 10
D = 6
M = B * S
L = 16

NC = 2
NS = 16
NW = NC * NS
NBG = 16
NST = S // 8
SLAB = 1024 * S // 4
DH = D // 2


def _sc_seq_body(idx_hbm, tab_hbm, out_hbm, slab, ob0, ob1, tab_v,
                 sem_s, sem_o0, sem_o1):
    wid = lax.axis_index("s") * NC + lax.axis_index("c")
    bg = lax.shift_right_logical(wid, 1)
    dh = lax.bitwise_and(wid, 1)
    pltpu.sync_copy(tab_hbm, tab_v)
    pltpu.async_copy(idx_hbm.at[pl.ds(bg * SLAB, SLAB)], slab, sem_s).wait()

    iota = lax.iota(jnp.int32, L)
    kp = lax.shift_right_logical(iota, 2) * S
    shv = lax.bitwise_and(iota, 3) * 8
    d_base = dh * (DH * L)

    obs = (ob0, ob1)
    sems = (sem_o0, sem_o1)
    out_h = [None, None]
    for job in range(DH * NST):
        dd, st = divmod(job, NST)
        ob, sem = obs[job % 2], sems[job % 2]
        if job >= 2:
            out_h[job % 2].wait()
        d16 = d_base + dd * L
        st8 = st * 8

        def body(t, c, ob=ob, d16=d16, st8=st8):
            s_in = lax.shift_right_logical(t, 6)
            u = lax.bitwise_and(t, 63)
            addr = (u * (4 * S) + (st8 + s_in)) + kp
            packed = plsc.load_gather(slab, [addr])
            e = lax.bitwise_and(
                lax.shift_right_logical(packed, shv), 255)
            v = plsc.load_gather(tab_v, [e + d16])
            ob[s_in, pl.ds(u * L, L)] = v
            return c

        lax.fori_loop(0, 512, body, 0, unroll=8)
        out_h[job % 2] = pltpu.async_copy(
            ob, out_hbm.at[dh * DH + dd, st, bg, :, :], sem)
    out_h[0].wait()
    out_h[1].wait()


_sc_seq = functools.partial(
    pl.kernel,
    mesh=plsc.VectorSubcoreMesh(core_axis_name="c", subcore_axis_name="s"),
    out_type=jax.ShapeDtypeStruct((D, NST, NBG, 8, 1024), jnp.float32),
    scratch_types=[
        pltpu.VMEM((SLAB,), jnp.int32),
        pltpu.VMEM((8, 1024), jnp.float32),
        pltpu.VMEM((8, 1024), jnp.float32),
        pltpu.VMEM((128,), jnp.float32),
        pltpu.SemaphoreType.DMA,
        pltpu.SemaphoreType.DMA,
        pltpu.SemaphoreType.DMA,
    ],
    compiler_params=pltpu.CompilerParams(needs_layout_passes=False),
)(_sc_seq_body)


_PB = 1024


def _pool_body(idx_ref, tp_ref, wp_ref, bp_ref, out_ref):
    n = jnp.dot(tp_ref[...], wp_ref[...], preferred_element_type=jnp.float32)
    n = n + bp_ref[...]
    idx = idx_ref[...]
    acc = jnp.zeros((_PB, D), jnp.float32)
    for v in range(V):
        acc = jnp.where(idx == v, n[v, :D], acc)
    out_ref[...] = jnp.tanh(acc)


def _pooled(idx0, tp, wp, bp):
    return pl.pallas_call(
        _pool_body,
        grid=(B // _PB,),
        in_specs=[
            pl.BlockSpec((_PB, 1), lambda i: (i, 0)),
            pl.BlockSpec((16, 128), lambda i: (0, 0)),
            pl.BlockSpec((128, 128), lambda i: (0, 0)),
            pl.BlockSpec((16, 128), lambda i: (0, 0)),
        ],
        out_specs=pl.BlockSpec((_PB, D), lambda i: (i, 0)),
        out_shape=jax.ShapeDtypeStruct((B, D), jnp.float32),
    )(idx0, tp, wp, bp)


def kernel(inputs, table, W, b):
    i4 = inputs.reshape(B // 4, 4, S)
    packed = (i4[:, 0] | (i4[:, 1] << 8) | (i4[:, 2] << 16)
              | (i4[:, 3] << 24)).reshape(M // 4)
    tab1d = jnp.zeros((8, L), jnp.float32).at[:D, :V].set(table.T).reshape(128)
    out6 = _sc_seq(packed, tab1d)
    seq = (out6.reshape(D, NST, NBG, 8, 8, 128)
           .transpose(2, 4, 5, 1, 3, 0)
           .reshape(B, S, D))

    idx0 = inputs[:, :1]
    tp = jnp.zeros((16, 128), jnp.float32).at[:V, :D].set(table)
    wp = jnp.zeros((128, 128), jnp.float32).at[:D, :D].set(W)
    bp = jnp.zeros((16, 128), jnp.float32).at[:, :D].set(b)
    pooled = _pooled(idx0, tp, wp, bp)
    return (seq, pooled)


# speedup vs baseline: 87.1496x; 3.4536x over previous
import functools

import jax
import jax.numpy as jnp
from jax import lax
from jax.experimental import pallas as pl
from jax.experimental.pallas import tpu as pltpu
from jax.experimental.pallas import tpu_sc as plsc

B = 16384
S = 200
V = 10
D = 6
M = B * S
L = 16

NC = 2
NS = 16
NW = NC * NS
NBG = 16
NST = S // 8
SLAB = 1024 * S // 4
DH = D // 2


def _sc_seq_body(idx_hbm, tab_hbm, out_hbm, slab, ob0, ob1, tab_v,
                 sem_s, sem_o0, sem_o1):
    wid = lax.axis_index("s") * NC + lax.axis_index("c")
    bg = lax.shift_right_logical(wid, 1)
    dh = lax.bitwise_and(wid, 1)
    pltpu.sync_copy(tab_hbm, tab_v)
    pltpu.async_copy(idx_hbm.at[pl.ds(bg * SLAB, SLAB)], slab, sem_s).wait()

    iota = lax.iota(jnp.int32, L)
    kp = lax.shift_right_logical(iota, 2) * S
    shv = lax.bitwise_and(iota, 3) * 8
    d_base = dh * (DH * L)

    obs = (ob0, ob1)
    sems = (sem_o0, sem_o1)
    out_h = [None, None]
    for job in range(DH * NST):
        dd, st = divmod(job, NST)
        ob, sem = obs[job % 2], sems[job % 2]
        if job >= 2:
            out_h[job % 2].wait()
        d16 = d_base + dd * L
        st8 = st * 8

        @plsc.parallel_loop(0, 512, unroll=8)
        def body(t, ob=ob, d16=d16, st8=st8):
            s_in = lax.shift_right_logical(t, 6)
            u = lax.bitwise_and(t, 63)
            addr = (u * (4 * S) + (st8 + s_in)) + kp
            packed = plsc.load_gather(slab, [addr])
            e = lax.bitwise_and(
                lax.shift_right_logical(packed, shv), 255)
            v = plsc.load_gather(tab_v, [e + d16])
            ob[s_in, pl.ds(u * L, L)] = v
        out_h[job % 2] = pltpu.async_copy(
            ob, out_hbm.at[dh * DH + dd, st, bg, :, :], sem)
    out_h[0].wait()
    out_h[1].wait()


_sc_seq = functools.partial(
    pl.kernel,
    mesh=plsc.VectorSubcoreMesh(core_axis_name="c", subcore_axis_name="s"),
    out_type=jax.ShapeDtypeStruct((D, NST, NBG, 8, 1024), jnp.float32),
    scratch_types=[
        pltpu.VMEM((SLAB,), jnp.int32),
        pltpu.VMEM((8, 1024), jnp.float32),
        pltpu.VMEM((8, 1024), jnp.float32),
        pltpu.VMEM((128,), jnp.float32),
        pltpu.SemaphoreType.DMA,
        pltpu.SemaphoreType.DMA,
        pltpu.SemaphoreType.DMA,
    ],
    compiler_params=pltpu.CompilerParams(needs_layout_passes=False),
)(_sc_seq_body)


_PB = 1024


def _pool_body(idx_ref, tp_ref, wp_ref, bp_ref, out_ref):
    n = jnp.dot(tp_ref[...], wp_ref[...], preferred_element_type=jnp.float32)
    n = n + bp_ref[...]
    idx = idx_ref[...]
    acc = jnp.zeros((_PB, D), jnp.float32)
    for v in range(V):
        acc = jnp.where(idx == v, n[v, :D], acc)
    out_ref[...] = jnp.tanh(acc)


def _pooled(idx0, tp, wp, bp):
    return pl.pallas_call(
        _pool_body,
        grid=(B // _PB,),
        in_specs=[
            pl.BlockSpec((_PB, 1), lambda i: (i, 0)),
            pl.BlockSpec((16, 128), lambda i: (0, 0)),
            pl.BlockSpec((128, 128), lambda i: (0, 0)),
            pl.BlockSpec((16, 128), lambda i: (0, 0)),
        ],
        out_specs=pl.BlockSpec((_PB, D), lambda i: (i, 0)),
        out_shape=jax.ShapeDtypeStruct((B, D), jnp.float32),
    )(idx0, tp, wp, bp)


def kernel(inputs, table, W, b):
    i4 = inputs.reshape(B // 4, 4, S)
    packed = (i4[:, 0] | (i4[:, 1] << 8) | (i4[:, 2] << 16)
              | (i4[:, 3] << 24)).reshape(M // 4)
    tab1d = jnp.zeros((8, L), jnp.float32).at[:D, :V].set(table.T).reshape(128)
    out6 = _sc_seq(packed, tab1d)
    seq = (out6.reshape(D, NST, NBG, 8, 8, 128)
           .transpose(2, 4, 5, 1, 3, 0)
           .reshape(B, S, D))

    idx0 = inputs[:, :1]
    tp = jnp.zeros((16, 128), jnp.float32).at[:V, :D].set(table)
    wp = jnp.zeros((128, 128), jnp.float32).at[:D, :D].set(W)
    bp = jnp.zeros((16, 128), jnp.float32).at[:, :D].set(b)
    pooled = _pooled(idx0, tp, wp, bp)
    return (seq, pooled)


# speedup vs baseline: 96.6450x; 1.1090x over previous
import functools

import jax
import jax.numpy as jnp
from jax import lax
from jax.experimental import pallas as pl
from jax.experimental.pallas import tpu as pltpu
from jax.experimental.pallas import tpu_sc as plsc

B = 16384
S = 200
V = 10
D = 6
M = B * S
L = 16

NC = 2
NS = 16
NW = NC * NS
NBG = 16
NST = S // 8
SLAB = 1024 * S // 4
DH = D // 2


def _sc_seq_body(idx_hbm, tab_hbm, out_hbm,
                 slab, ob00, ob01, ob02, ob10, ob11, ob12, tab_v,
                 sem_s, sem_o0, sem_o1, sem_o2, sem_o3, sem_o4, sem_o5):
    wid = lax.axis_index("s") * NC + lax.axis_index("c")
    bg = lax.shift_right_logical(wid, 1)
    dh = lax.bitwise_and(wid, 1)
    pltpu.sync_copy(tab_hbm, tab_v)
    pltpu.async_copy(idx_hbm.at[pl.ds(bg * SLAB, SLAB)], slab, sem_s).wait()

    iota = lax.iota(jnp.int32, L)
    kp = lax.shift_right_logical(iota, 2) * S
    shv = lax.bitwise_and(iota, 3) * 8
    d_base = dh * (DH * L)

    obs = ((ob00, ob01, ob02), (ob10, ob11, ob12))
    sems = ((sem_o0, sem_o1, sem_o2), (sem_o3, sem_o4, sem_o5))
    out_h = [[None] * DH, [None] * DH]
    for st in range(NST):
        par = st % 2
        ob = obs[par]
        if st >= 2:
            for dd in range(DH):
                out_h[par][dd].wait()
        st8 = st * 8

        @plsc.parallel_loop(0, 512, unroll=8)
        def body(t, ob=ob, st8=st8):
            s_in = lax.shift_right_logical(t, 6)
            u = lax.bitwise_and(t, 63)
            addr = (u * (4 * S) + (st8 + s_in)) + kp
            packed = plsc.load_gather(slab, [addr])
            e = lax.bitwise_and(
                lax.shift_right_logical(packed, shv), 255) + d_base
            for dd in range(DH):
                v = plsc.load_gather(tab_v, [e + dd * L])
                ob[dd][s_in, pl.ds(u * L, L)] = v

        for dd in range(DH):
            out_h[par][dd] = pltpu.async_copy(
                ob[dd], out_hbm.at[dh * DH + dd, st, bg, :, :],
                sems[par][dd])
    for par in range(2):
        for dd in range(DH):
            out_h[par][dd].wait()


_sc_seq = functools.partial(
    pl.kernel,
    mesh=plsc.VectorSubcoreMesh(core_axis_name="c", subcore_axis_name="s"),
    out_type=jax.ShapeDtypeStruct((D, NST, NBG, 8, 1024), jnp.float32),
    scratch_types=[
        pltpu.VMEM((SLAB,), jnp.int32),
        pltpu.VMEM((8, 1024), jnp.float32),
        pltpu.VMEM((8, 1024), jnp.float32),
        pltpu.VMEM((8, 1024), jnp.float32),
        pltpu.VMEM((8, 1024), jnp.float32),
        pltpu.VMEM((8, 1024), jnp.float32),
        pltpu.VMEM((8, 1024), jnp.float32),
        pltpu.VMEM((128,), jnp.float32),
        pltpu.SemaphoreType.DMA,
        pltpu.SemaphoreType.DMA,
        pltpu.SemaphoreType.DMA,
        pltpu.SemaphoreType.DMA,
        pltpu.SemaphoreType.DMA,
        pltpu.SemaphoreType.DMA,
        pltpu.SemaphoreType.DMA,
    ],
    compiler_params=pltpu.CompilerParams(needs_layout_passes=False),
)(_sc_seq_body)


_PB = 1024


def _pool_body(idx_ref, tp_ref, wp_ref, bp_ref, out_ref):
    n = jnp.dot(tp_ref[...], wp_ref[...], preferred_element_type=jnp.float32)
    n = n + bp_ref[...]
    idx = idx_ref[...]
    acc = jnp.zeros((_PB, D), jnp.float32)
    for v in range(V):
        acc = jnp.where(idx == v, n[v, :D], acc)
    out_ref[...] = jnp.tanh(acc)


def _pooled(idx0, tp, wp, bp):
    return pl.pallas_call(
        _pool_body,
        grid=(B // _PB,),
        in_specs=[
            pl.BlockSpec((_PB, 1), lambda i: (i, 0)),
            pl.BlockSpec((16, 128), lambda i: (0, 0)),
            pl.BlockSpec((128, 128), lambda i: (0, 0)),
            pl.BlockSpec((16, 128), lambda i: (0, 0)),
        ],
        out_specs=pl.BlockSpec((_PB, D), lambda i: (i, 0)),
        out_shape=jax.ShapeDtypeStruct((B, D), jnp.float32),
    )(idx0, tp, wp, bp)


def kernel(inputs, table, W, b):
    i4 = inputs.reshape(B // 4, 4, S)
    packed = (i4[:, 0] | (i4[:, 1] << 8) | (i4[:, 2] << 16)
              | (i4[:, 3] << 24)).reshape(M // 4)
    tab1d = jnp.zeros((8, L), jnp.float32).at[:D, :V].set(table.T).reshape(128)
    out6 = _sc_seq(packed, tab1d)
    seq = (out6.reshape(D, NST, NBG, 8, 8, 128)
           .transpose(2, 4, 5, 1, 3, 0)
           .reshape(B, S, D))

    idx0 = inputs[:, :1]
    tp = jnp.zeros((16, 128), jnp.float32).at[:V, :D].set(table)
    wp = jnp.zeros((128, 128), jnp.float32).at[:D, :D].set(W)
    bp = jnp.zeros((16, 128), jnp.float32).at[:, :D].set(b)
    pooled = _pooled(idx0, tp, wp, bp)
    return (seq, pooled)


# speedup vs baseline: 176.9131x; 1.8305x over previous
import functools

import jax
import jax.numpy as jnp
from jax import lax
from jax.experimental import pallas as pl
from jax.experimental.pallas import tpu as pltpu
from jax.experimental.pallas import tpu_sc as plsc

B = 16384
S = 200
V = 10
D = 6
L = 16

NC = 2
NBG = 16
NST = S // 8
DH = D // 2
CW = 8 * 8 * 128


def _sc_seq_body(idx_hbm, tab_hbm, out_hbm,
                 ich0, ich1, ob00, ob01, ob02, ob10, ob11, ob12, tab_v,
                 sem_i0, sem_i1, sem_o0, sem_o1, sem_o2, sem_o3, sem_o4,
                 sem_o5):
    wid = lax.axis_index("s") * NC + lax.axis_index("c")
    bg = lax.shift_right_logical(wid, 1)
    dh = lax.bitwise_and(wid, 1)
    pltpu.sync_copy(tab_hbm, tab_v)
    d_base = dh * (DH * L)

    ichs = (ich0, ich1)
    isems = (sem_i0, sem_i1)
    obs = ((ob00, ob01, ob02), (ob10, ob11, ob12))
    osems = ((sem_o0, sem_o1, sem_o2), (sem_o3, sem_o4, sem_o5))

    in_h = [None, None]
    out_h = [[None] * DH, [None] * DH]
    in_h[0] = pltpu.async_copy(idx_hbm.at[0, bg, :, :], ichs[0], isems[0])
    for st in range(NST):
        par = st % 2
        if st + 1 < NST:
            in_h[1 - par] = pltpu.async_copy(
                idx_hbm.at[st + 1, bg, :, :], ichs[1 - par], isems[1 - par])
        in_h[par].wait()
        ich = ichs[par]
        ob = obs[par]
        if st >= 2:
            for dd in range(DH):
                out_h[par][dd].wait()

        @plsc.parallel_loop(0, CW // L, unroll=8)
        def body(t, ich=ich, ob=ob):
            r = lax.shift_right_logical(t, 6)
            c = lax.bitwise_and(t, 63) * L
            e = ich[r, pl.ds(c, L)] + d_base
            for dd in range(DH):
                v = plsc.load_gather(tab_v, [e + dd * L])
                ob[dd][r, pl.ds(c, L)] = v

        for dd in range(DH):
            out_h[par][dd] = pltpu.async_copy(
                ob[dd], out_hbm.at[dh * DH + dd, st, bg, :, :],
                osems[par][dd])
    for par in range(2):
        for dd in range(DH):
            out_h[par][dd].wait()


_sc_seq = functools.partial(
    pl.kernel,
    mesh=plsc.VectorSubcoreMesh(core_axis_name="c", subcore_axis_name="s"),
    out_type=jax.ShapeDtypeStruct((D, NST, NBG, 8, 1024), jnp.float32),
    scratch_types=[
        pltpu.VMEM((8, 1024), jnp.int32),
        pltpu.VMEM((8, 1024), jnp.int32),
        pltpu.VMEM((8, 1024), jnp.float32),
        pltpu.VMEM((8, 1024), jnp.float32),
        pltpu.VMEM((8, 1024), jnp.float32),
        pltpu.VMEM((8, 1024), jnp.float32),
        pltpu.VMEM((8, 1024), jnp.float32),
        pltpu.VMEM((8, 1024), jnp.float32),
        pltpu.VMEM((128,), jnp.float32),
        pltpu.SemaphoreType.DMA,
        pltpu.SemaphoreType.DMA,
        pltpu.SemaphoreType.DMA,
        pltpu.SemaphoreType.DMA,
        pltpu.SemaphoreType.DMA,
        pltpu.SemaphoreType.DMA,
        pltpu.SemaphoreType.DMA,
        pltpu.SemaphoreType.DMA,
    ],
    compiler_params=pltpu.CompilerParams(needs_layout_passes=False),
)(_sc_seq_body)


_PB = 1024


def _pool_body(idx_ref, tp_ref, wp_ref, bp_ref, out_ref):
    n = jnp.dot(tp_ref[...], wp_ref[...], preferred_element_type=jnp.float32)
    n = n + bp_ref[...]
    idx = idx_ref[...]
    acc = jnp.zeros((_PB, D), jnp.float32)
    for v in range(V):
        acc = jnp.where(idx == v, n[v, :D], acc)
    out_ref[...] = jnp.tanh(acc)


def _pooled(idx0, tp, wp, bp):
    return pl.pallas_call(
        _pool_body,
        grid=(B // _PB,),
        in_specs=[
            pl.BlockSpec((_PB, 1), lambda i: (i, 0)),
            pl.BlockSpec((16, 128), lambda i: (0, 0)),
            pl.BlockSpec((128, 128), lambda i: (0, 0)),
            pl.BlockSpec((16, 128), lambda i: (0, 0)),
        ],
        out_specs=pl.BlockSpec((_PB, D), lambda i: (i, 0)),
        out_shape=jax.ShapeDtypeStruct((B, D), jnp.float32),
    )(idx0, tp, wp, bp)


def kernel(inputs, table, W, b):
    idx5 = (inputs.reshape(NBG, 8, 128, NST, 8)
            .transpose(3, 0, 4, 1, 2)
            .reshape(NST, NBG, 8, 1024))
    tab1d = jnp.zeros((8, L), jnp.float32).at[:D, :V].set(table.T).reshape(128)
    out6 = _sc_seq(idx5, tab1d)
    seq = (out6.reshape(D, NST, NBG, 8, 8, 128)
           .transpose(2, 4, 5, 1, 3, 0)
           .reshape(B, S, D))

    idx0 = inputs[:, :1]
    tp = jnp.zeros((16, 128), jnp.float32).at[:V, :D].set(table)
    wp = jnp.zeros((128, 128), jnp.float32).at[:D, :D].set(W)
    bp = jnp.zeros((16, 128), jnp.float32).at[:, :D].set(b)
    pooled = _pooled(idx0, tp, wp, bp)
    return (seq, pooled)
